# baseline (speedup 1.0000x reference)
import functools

import jax
import jax.numpy as jnp
from jax import lax
from jax.experimental import pallas as pl
from jax.experimental.pallas import tpu as pltpu
from jax.experimental.pallas import tpu_sc as plsc

f32 = jnp.float32
i32 = jnp.int32

_N = 10000
_E = 320000
_P = 128
_H = 256
_C = 16
_G = 128
_BLK = 128
_TH = 5120
_NCH = 7
_CB = 12
_NB = _NCH * _CB
_CAP = 16 * _NB * _BLK
_AR = 5248
_ZR = _AR // 16
_NBROW = 400
_GRID = _N // _NBROW

def _sc_edge_pass_body(tbl_hbm, edgeI0_hbm, edgeI1_hbm,
                       out_hbm, ich0, ich1, buf,
                       isem0, isem1, acc):
    c = lax.axis_index("c")
    s = lax.axis_index("s")

    @pl.loop(0, _BLK)
    def _(r):
        @pl.loop(0, 2)
        def _(h):
            @pl.loop(0, 8)
            def _(k):
                buf[r, h, pl.ds(k * 16, 16)] = jnp.zeros((16,), f32)

    @pl.loop(0, 2)
    def _(k):
        pltpu.sync_copy(buf, acc.at[pl.ds(s * _ZR + k * _BLK, _BLK)])

    pltpu.sync_copy(buf.at[pl.ds(0, _ZR - 2 * _BLK)],
                    acc.at[pl.ds(s * _ZR + 2 * _BLK, _ZR - 2 * _BLK)])
    plsc.subcore_barrier()

    def work(edgeI_hbm):
        me = edgeI_hbm.at[s]

        def ifetch(cc, ich, isem):
            pltpu.async_copy(me.at[cc], ich, isem)

        def iwait(ich, isem):
            pltpu.make_async_copy(me.at[0], ich, isem).wait()

        def chunk(ich):
            @pl.loop(0, _CB)
            def _(k):
                pltpu.sync_copy(tbl_hbm.at[ich.at[k].at[0]], buf)
                pltpu.sync_copy(buf, acc.at[ich.at[k].at[1]], add=True)

        ifetch(0, ich0, isem0)
        iwait(ich0, isem0)
        ifetch(1, ich1, isem1)

        @pl.loop(0, _NCH, step=2)
        def _(cc):
            chunk(ich0)

            @pl.when(cc + 2 < _NCH)
            def _():
                ifetch(cc + 2, ich0, isem0)

            @pl.when(cc + 1 < _NCH)
            def _():
                iwait(ich1, isem1)
                chunk(ich1)

                @pl.when(cc + 3 < _NCH)
                def _():
                    ifetch(cc + 3, ich1, isem1)

                @pl.when(cc + 2 < _NCH)
                def _():
                    iwait(ich0, isem0)

    @pl.when(c == 0)
    def _():
        work(edgeI0_hbm)

    @pl.when(c == 1)
    def _():
        work(edgeI1_hbm)

    plsc.subcore_barrier()
    pltpu.sync_copy(acc.at[pl.ds(s * _ZR, _ZR)],
                    out_hbm.at[pl.ds(c * _AR + s * _ZR, _ZR)])


@functools.lru_cache(maxsize=None)
def _sc_kernels():
    mesh = plsc.VectorSubcoreMesh(core_axis_name="c", subcore_axis_name="s")
    sc_edge_pass = pl.kernel(
        _sc_edge_pass_body,
        out_type=jax.ShapeDtypeStruct((2 * _AR, 2, 128), f32),
        mesh=mesh,
        scratch_types=[pltpu.VMEM((_CB, 2, _BLK), i32),
                       pltpu.VMEM((_CB, 2, _BLK), i32),
                       pltpu.VMEM((_BLK, 2, 128), f32),
                       pltpu.SemaphoreType.DMA,
                       pltpu.SemaphoreType.DMA,
                       pltpu.VMEM_SHARED((_AR, 2, 128), f32)])
    return sc_edge_pass


def _row(i):
    return (i, 0)


def _const(i):
    return (0, 0)


def _rspec(cols):
    return pl.BlockSpec((_NBROW, cols), _row)


def _cspec(shape):
    return pl.BlockSpec(shape, _const)


def _dinv_of(dA, dB):
    deg = dA[:, :1] + dB[:, :1]
    return lax.rsqrt(jnp.maximum(deg, 1.0))


def _prepare_body(x_ref, xs_ref, dA_ref, dB_ref, root_ref,
                  wpf_ref, bpf_ref, wpc_ref, bpc_ref,
                  wco_ref, bco_ref, wad_ref, bad_ref,
                  hF_ref, ctrl_ref, cond_ref, gf_ref, gc_ref):
    i = pl.program_id(0)
    xb = x_ref[...]
    dinv = _dinv_of(dA_ref[...], dB_ref[...])
    rowid = i * _NBROW + lax.broadcasted_iota(i32, (_NBROW, _G), 0)
    bump = jnp.sum((rowid == root_ref[...]).astype(f32), axis=1, keepdims=True)
    hf = jnp.maximum(
        jnp.dot(xb, wpf_ref[...], preferred_element_type=f32)
        + bpf_ref[...] + bump, 0.0)
    hc = jnp.maximum(
        jnp.dot(xb, wpc_ref[...], preferred_element_type=f32)
        + bpc_ref[...] + bump, 0.0)
    cond = (jnp.dot(xs_ref[...], wco_ref[...], preferred_element_type=f32)
            + bco_ref[...])
    adapt = (jnp.dot(cond, wad_ref[...], preferred_element_type=f32)
             + bad_ref[...])
    ctrl = hc + adapt
    gf = dinv * hf
    gc = dinv * ctrl
    hF_ref[...] = hf
    ctrl_ref[...] = ctrl
    cond_ref[...] = cond
    gf_ref[...] = gf
    gc_ref[...] = gc


_prepare = pl.pallas_call(
    _prepare_body,
    grid=(_GRID,),
    in_specs=[_rspec(_P), _rspec(_P), _rspec(16), _rspec(16),
              _cspec((1, _G)),
              _cspec((_P, _H)), _cspec((1, _H)),
              _cspec((_P, _H)), _cspec((1, _H)),
              _cspec((_P, _H)), _cspec((1, _H)),
              _cspec((_H, _H)), _cspec((1, _H))],
    out_specs=[_rspec(_H)] * 5,
    out_shape=[jax.ShapeDtypeStruct((_N, _H), f32)] * 5,
)


def _layer_body(last, af_ref, ac_ref,
                hf_ref, ctrl_ref, cond_ref, hsum_ref, dA_ref, dB_ref,
                wf_ref, bf_ref, wc_ref, bc_ref, wz_ref, bz_ref,
                *out_refs):
    dinv = _dinv_of(dA_ref[...], dB_ref[...])
    aggF = dinv * af_ref[...]
    aggC = dinv * ac_ref[...]
    hfmid = jnp.maximum(
        jnp.dot(aggF + hf_ref[...], wf_ref[...], preferred_element_type=f32)
        + bf_ref[...], 0.0)
    hcn = jnp.maximum(
        jnp.dot(aggC + ctrl_ref[...], wc_ref[...], preferred_element_type=f32)
        + bc_ref[...], 0.0)
    hfn = hfmid + jnp.dot(hcn, wz_ref[...], preferred_element_type=f32) \
        + bz_ref[...]
    hsumn = hsum_ref[...] + hfn
    if last:
        (hsum_out,) = out_refs
        hsum_out[...] = hsumn
        return
    hf_out, ctrl_out, hsum_out, gf_ref, gc_ref = out_refs
    ctrln = hcn + cond_ref[...]
    hf_out[...] = hfn
    ctrl_out[...] = ctrln
    hsum_out[...] = hsumn
    gf_ref[...] = dinv * hfn
    gc_ref[...] = dinv * ctrln


_layer_in_specs = [_rspec(_H)] * 6 + [_rspec(16)] * 2 \
    + [_cspec((_H, _H)), _cspec((1, _H))] * 3

_layer_mid = pl.pallas_call(
    functools.partial(_layer_body, False),
    grid=(_GRID,),
    in_specs=_layer_in_specs,
    out_specs=[_rspec(_H)] * 5,
    out_shape=[jax.ShapeDtypeStruct((_N, _H), f32)] * 5,
)

_layer_last = pl.pallas_call(
    functools.partial(_layer_body, True),
    grid=(_GRID,),
    in_specs=_layer_in_specs,
    out_specs=[_rspec(_H)],
    out_shape=[jax.ShapeDtypeStruct((_N, _H), f32)],
)


def _readout_body(h_ref, b_ref, wcls_ref, bcls_ref, out_ref, acc_ref, cnt_ref):
    i = pl.program_id(0)
    bid = b_ref[0, 0, :]
    onehot = (bid[:, None]
              == lax.broadcasted_iota(i32, (_NBROW, _G), 1)).astype(f32)
    part = lax.dot_general(onehot, h_ref[...], (((0,), (0,)), ((), ())),
                           preferred_element_type=f32)

    @pl.when(i == 0)
    def _():
        acc_ref[...] = jnp.zeros_like(acc_ref)
        cnt_ref[...] = jnp.zeros_like(cnt_ref)

    acc_ref[...] += part
    cnt_ref[0:1, :] = cnt_ref[0:1, :] + jnp.sum(onehot, axis=0)[None, :]

    @pl.when(i == _GRID - 1)
    def _():
        cnt = jnp.maximum(cnt_ref[0, :], 1.0)
        o = acc_ref[...] / cnt[:, None]
        nrm = jnp.sqrt(jnp.sum(o * o, axis=1, keepdims=True))
        o = o / jnp.maximum(nrm, 1e-5)
        out_ref[...] = (jnp.dot(o, wcls_ref[...], preferred_element_type=f32)
                        + bcls_ref[...])


_readout = pl.pallas_call(
    _readout_body,
    grid=(_GRID,),
    in_specs=[_rspec(_H),
              pl.BlockSpec((1, 1, _NBROW), lambda i: (i, 0, 0)),
              _cspec((_H, 128)), _cspec((1, 128))],
    out_specs=[_cspec((_G, 128))],
    out_shape=[jax.ShapeDtypeStruct((_G, 128), f32)],
    scratch_shapes=[pltpu.VMEM((_G, _H), f32), pltpu.VMEM((8, _G), f32)],
)


def kernel(x, x_sim, edge_index, batch, root_n_id, frozen,
           Wp_f, bp_f, Wp_c, bp_c,
           Wf0, bf0, Wf1, bf1, Wf2, bf2,
           Wc0, bc0, Wc1, bc1, Wc2, bc2,
           Wz0, bz0, Wz1, bz1, Wz2, bz2,
           Wcond, bcond, Wadapt, badapt, Wcls, bcls):
    del frozen
    src = edge_index[0]
    dst = edge_index[1]
    key = (dst >= _TH).astype(i32)
    perm = jnp.argsort(key, stable=True)
    srcs = src[perm]
    dsts = dst[perm]
    cnt0 = _E - jnp.sum(key)
    ii = jnp.arange(_CAP, dtype=i32)
    m0 = ii < cnt0
    src0 = jnp.where(m0, srcs[:_CAP], 0)
    dst0 = jnp.where(m0, dsts[:_CAP], _TH)
    i1 = jnp.clip(cnt0 + ii, 0, _E - 1)
    m1 = ii < (_E - cnt0)
    src1 = jnp.where(m1, srcs[i1], 0)
    dst1 = jnp.where(m1, dsts[i1] - _TH, _N - _TH)

    def _pack(s_, d_):
        sI = s_.reshape(16, _NCH, _CB, 128)
        dI = d_.reshape(16, _NCH, _CB, 128)
        return jnp.stack([sI, dI], axis=3)

    edgeI0 = _pack(src0, dst0)
    edgeI1 = _pack(src1, dst1)

    def _assemble(out):
        return jnp.concatenate([out[:_TH], out[_AR:_AR + (_N - _TH)]])

    _sc_edge_pass = _sc_kernels()

    def edge_pass(table):
        out = _sc_edge_pass(table.reshape(_N, 2, 128), edgeI0, edgeI1)
        return _assemble(out.reshape(2 * _AR, _H))

    onesT = jnp.ones((_N, _H), f32)
    degA = edge_pass(onesT)[:, :16]
    degB = jnp.zeros((_N, 16), f32)

    r2 = lambda b: b.reshape(1, -1)
    root2 = root_n_id.reshape(1, _G)
    hF, ctrl, cond, gF, gC = _prepare(
        x, x_sim, degA, degB, root2,
        Wp_f, r2(bp_f), Wp_c, r2(bp_c),
        Wcond, r2(bcond), Wadapt, r2(badapt))
    hsum = hF

    Wf = (Wf0, Wf1, Wf2)
    bf = (bf0, bf1, bf2)
    Wc = (Wc0, Wc1, Wc2)
    bc = (bc0, bc1, bc2)
    Wz = (Wz0, Wz1, Wz2)
    bz = (bz0, bz1, bz2)
    for l in range(3):
        aF = edge_pass(gF)
        aC = edge_pass(gC)
        args = (aF, aC, hF, ctrl, cond, hsum, degA, degB,
                Wf[l], r2(bf[l]), Wc[l], r2(bc[l]), Wz[l], r2(bz[l]))
        if l < 2:
            hF, ctrl, hsum, gF, gC = _layer_mid(*args)
        else:
            (hsum,) = _layer_last(*args)

    wcls_pad = jnp.pad(Wcls, ((0, 0), (0, 128 - _C)))
    bcls_pad = jnp.pad(bcls, ((0, 128 - _C),)).reshape(1, 128)
    batch3 = batch.reshape(_GRID, 1, _NBROW)
    (outp,) = _readout(hsum, batch3, wcls_pad, bcls_pad)
    return outp[:, :_C]


# speedup vs baseline: 2.2152x; 2.2152x over previous
import functools

import jax
import jax.numpy as jnp
from jax import lax
from jax.experimental import pallas as pl
from jax.experimental.pallas import tpu as pltpu
from jax.experimental.pallas import tpu_sc as plsc

f32 = jnp.float32
i32 = jnp.int32

_N = 10000
_E = 320000
_P = 128
_H = 256
_C = 16
_G = 128
_BLK = 128
_EPAD = 2528 * _BLK
_NB = 79
_NPAD = 10112
_ZR = _NPAD // 16
_NBROW = 400
_GRID = _N // _NBROW


def _sc_edge_pass_body(tA_hbm, tB_hbm, srcI_hbm, dstI_hbm, z128_hbm,
                       outA_hbm, outB_hbm, src_v, dst_v, buf, acc):
    c = lax.axis_index("c")
    s = lax.axis_index("s")
    pltpu.sync_copy(z128_hbm, acc.at[pl.ds(s * _ZR, _ZR)])
    pltpu.sync_copy(srcI_hbm.at[s], src_v)
    pltpu.sync_copy(dstI_hbm.at[s], dst_v)
    plsc.subcore_barrier()

    @pl.when(c == 0)
    def _():
        @pl.loop(0, _NB)
        def _(j):
            pltpu.sync_copy(tA_hbm.at[src_v.at[j]], buf)
            pltpu.sync_copy(buf, acc.at[dst_v.at[j]], add=True)

    @pl.when(c == 1)
    def _():
        @pl.loop(0, _NB)
        def _(j):
            pltpu.sync_copy(tB_hbm.at[src_v.at[j]], buf)
            pltpu.sync_copy(buf, acc.at[dst_v.at[j]], add=True)

    plsc.subcore_barrier()

    @pl.when(c == 0)
    def _():
        pltpu.sync_copy(acc.at[pl.ds(s * _ZR, _ZR)],
                        outA_hbm.at[pl.ds(s * _ZR, _ZR)])

    @pl.when(c == 1)
    def _():
        pltpu.sync_copy(acc.at[pl.ds(s * _ZR, _ZR)],
                        outB_hbm.at[pl.ds(s * _ZR, _ZR)])


@functools.lru_cache(maxsize=None)
def _sc_kernels():
    mesh = plsc.VectorSubcoreMesh(core_axis_name="c", subcore_axis_name="s")
    sc_edge_pass = pl.kernel(
        _sc_edge_pass_body,
        out_type=[jax.ShapeDtypeStruct((_NPAD, 128), f32),
                  jax.ShapeDtypeStruct((_NPAD, 128), f32)],
        mesh=mesh,
        scratch_types=[pltpu.VMEM((_NB, _BLK), i32),
                       pltpu.VMEM((_NB, _BLK), i32),
                       pltpu.VMEM((_BLK, 128), f32),
                       pltpu.VMEM_SHARED((_NPAD, 128), f32)])
    return sc_edge_pass


def _row(i):
    return (i, 0)


def _const(i):
    return (0, 0)


def _rspec(cols):
    return pl.BlockSpec((_NBROW, cols), _row)


def _cspec(shape):
    return pl.BlockSpec(shape, _const)


def _dinv_of(dA, dB):
    deg = dA[:, :1] + dB[:, :1]
    return lax.rsqrt(jnp.maximum(deg, 1.0))


def _prepare_body(x_ref, xs_ref, dA_ref, dB_ref, root_ref,
                  wpf_ref, bpf_ref, wpc_ref, bpc_ref,
                  wco_ref, bco_ref, wad_ref, bad_ref,
                  hF_ref, ctrl_ref, cond_ref,
                  gfa_ref, gfb_ref, gca_ref, gcb_ref):
    i = pl.program_id(0)
    xb = x_ref[...]
    dinv = _dinv_of(dA_ref[...], dB_ref[...])
    rowid = i * _NBROW + lax.broadcasted_iota(i32, (_NBROW, _G), 0)
    bump = jnp.sum((rowid == root_ref[...]).astype(f32), axis=1, keepdims=True)
    hf = jnp.maximum(
        jnp.dot(xb, wpf_ref[...], preferred_element_type=f32)
        + bpf_ref[...] + bump, 0.0)
    hc = jnp.maximum(
        jnp.dot(xb, wpc_ref[...], preferred_element_type=f32)
        + bpc_ref[...] + bump, 0.0)
    cond = (jnp.dot(xs_ref[...], wco_ref[...], preferred_element_type=f32)
            + bco_ref[...])
    adapt = (jnp.dot(cond, wad_ref[...], preferred_element_type=f32)
             + bad_ref[...])
    ctrl = hc + adapt
    gf = dinv * hf
    gc = dinv * ctrl
    hF_ref[...] = hf
    ctrl_ref[...] = ctrl
    cond_ref[...] = cond
    gfa_ref[...] = gf[:, :128]
    gfb_ref[...] = gf[:, 128:]
    gca_ref[...] = gc[:, :128]
    gcb_ref[...] = gc[:, 128:]


_prepare = pl.pallas_call(
    _prepare_body,
    grid=(_GRID,),
    in_specs=[_rspec(_P), _rspec(_P), _rspec(16), _rspec(16),
              _cspec((1, _G)),
              _cspec((_P, _H)), _cspec((1, _H)),
              _cspec((_P, _H)), _cspec((1, _H)),
              _cspec((_P, _H)), _cspec((1, _H)),
              _cspec((_H, _H)), _cspec((1, _H))],
    out_specs=[_rspec(_H), _rspec(_H), _rspec(_H),
               _rspec(128), _rspec(128), _rspec(128), _rspec(128)],
    out_shape=[jax.ShapeDtypeStruct((_N, _H), f32)] * 3
    + [jax.ShapeDtypeStruct((_N, 128), f32)] * 4,
)


def _layer_body(last, afa0_ref, afb0_ref, afa1_ref, afb1_ref,
                aca0_ref, acb0_ref, aca1_ref, acb1_ref,
                hf_ref, ctrl_ref, cond_ref, hsum_ref, dA_ref, dB_ref,
                wf_ref, bf_ref, wc_ref, bc_ref, wz_ref, bz_ref,
                *out_refs):
    dinv = _dinv_of(dA_ref[...], dB_ref[...])
    aggF = dinv * jnp.concatenate([afa0_ref[...] + afa1_ref[...],
                                   afb0_ref[...] + afb1_ref[...]], axis=1)
    aggC = dinv * jnp.concatenate([aca0_ref[...] + aca1_ref[...],
                                   acb0_ref[...] + acb1_ref[...]], axis=1)
    hfmid = jnp.maximum(
        jnp.dot(aggF + hf_ref[...], wf_ref[...], preferred_element_type=f32)
        + bf_ref[...], 0.0)
    hcn = jnp.maximum(
        jnp.dot(aggC + ctrl_ref[...], wc_ref[...], preferred_element_type=f32)
        + bc_ref[...], 0.0)
    hfn = hfmid + jnp.dot(hcn, wz_ref[...], preferred_element_type=f32) \
        + bz_ref[...]
    hsumn = hsum_ref[...] + hfn
    if last:
        (hsum_out,) = out_refs
        hsum_out[...] = hsumn
        return
    hf_out, ctrl_out, hsum_out, gfa_ref, gfb_ref, gca_ref, gcb_ref = out_refs
    ctrln = hcn + cond_ref[...]
    gf = dinv * hfn
    gc = dinv * ctrln
    hf_out[...] = hfn
    ctrl_out[...] = ctrln
    hsum_out[...] = hsumn
    gfa_ref[...] = gf[:, :128]
    gfb_ref[...] = gf[:, 128:]
    gca_ref[...] = gc[:, :128]
    gcb_ref[...] = gc[:, 128:]


_layer_in_specs = [_rspec(128)] * 8 + [_rspec(_H)] * 4 + [_rspec(16)] * 2 \
    + [_cspec((_H, _H)), _cspec((1, _H))] * 3

_layer_mid = pl.pallas_call(
    functools.partial(_layer_body, False),
    grid=(_GRID,),
    in_specs=_layer_in_specs,
    out_specs=[_rspec(_H)] * 3 + [_rspec(128)] * 4,
    out_shape=[jax.ShapeDtypeStruct((_N, _H), f32)] * 3
    + [jax.ShapeDtypeStruct((_N, 128), f32)] * 4,
)

_layer_last = pl.pallas_call(
    functools.partial(_layer_body, True),
    grid=(_GRID,),
    in_specs=_layer_in_specs,
    out_specs=[_rspec(_H)],
    out_shape=[jax.ShapeDtypeStruct((_N, _H), f32)],
)


def _readout_body(h_ref, b_ref, wcls_ref, bcls_ref, out_ref, acc_ref, cnt_ref):
    i = pl.program_id(0)
    bid = b_ref[0, 0, :]
    onehot = (bid[:, None]
              == lax.broadcasted_iota(i32, (_NBROW, _G), 1)).astype(f32)
    part = lax.dot_general(onehot, h_ref[...], (((0,), (0,)), ((), ())),
                           preferred_element_type=f32)

    @pl.when(i == 0)
    def _():
        acc_ref[...] = jnp.zeros_like(acc_ref)
        cnt_ref[...] = jnp.zeros_like(cnt_ref)

    acc_ref[...] += part
    cnt_ref[0:1, :] = cnt_ref[0:1, :] + jnp.sum(onehot, axis=0)[None, :]

    @pl.when(i == _GRID - 1)
    def _():
        cnt = jnp.maximum(cnt_ref[0, :], 1.0)
        o = acc_ref[...] / cnt[:, None]
        nrm = jnp.sqrt(jnp.sum(o * o, axis=1, keepdims=True))
        o = o / jnp.maximum(nrm, 1e-5)
        out_ref[...] = (jnp.dot(o, wcls_ref[...], preferred_element_type=f32)
                        + bcls_ref[...])


_readout = pl.pallas_call(
    _readout_body,
    grid=(_GRID,),
    in_specs=[_rspec(_H),
              pl.BlockSpec((1, 1, _NBROW), lambda i: (i, 0, 0)),
              _cspec((_H, 128)), _cspec((1, 128))],
    out_specs=[_cspec((_G, 128))],
    out_shape=[jax.ShapeDtypeStruct((_G, 128), f32)],
    scratch_shapes=[pltpu.VMEM((_G, _H), f32), pltpu.VMEM((8, _G), f32)],
)


def kernel(x, x_sim, edge_index, batch, root_n_id, frozen,
           Wp_f, bp_f, Wp_c, bp_c,
           Wf0, bf0, Wf1, bf1, Wf2, bf2,
           Wc0, bc0, Wc1, bc1, Wc2, bc2,
           Wz0, bz0, Wz1, bz1, Wz2, bz2,
           Wcond, bcond, Wadapt, badapt, Wcls, bcls):
    del frozen
    src = edge_index[0]
    dst = edge_index[1]
    pad = _EPAD - _E
    srcp = jnp.concatenate([src, jnp.zeros((pad,), i32)])
    dstp = jnp.concatenate([dst, jnp.full((pad,), _N, i32)])
    srcI = srcp.reshape(2, 16, _NB, _BLK)
    dstI = dstp.reshape(2, 16, _NB, _BLK)
    z128 = jnp.zeros((_ZR, 128), f32)

    _sc_edge_pass = _sc_kernels()
    onesT = jnp.ones((_N, 128), f32)
    dT0, _unused0 = _sc_edge_pass(onesT, onesT, srcI[0], dstI[0], z128)
    dT1, _unused1 = _sc_edge_pass(onesT, onesT, srcI[1], dstI[1], z128)
    degA = dT0[:_N, :16]
    degB = dT1[:_N, :16]

    r2 = lambda b: b.reshape(1, -1)
    root2 = root_n_id.reshape(1, _G)
    hF, ctrl, cond, gFA, gFB, gCA, gCB = _prepare(
        x, x_sim, degA, degB, root2,
        Wp_f, r2(bp_f), Wp_c, r2(bp_c),
        Wcond, r2(bcond), Wadapt, r2(badapt))
    hsum = hF

    Wf = (Wf0, Wf1, Wf2)
    bf = (bf0, bf1, bf2)
    Wc = (Wc0, Wc1, Wc2)
    bc = (bc0, bc1, bc2)
    Wz = (Wz0, Wz1, Wz2)
    bz = (bz0, bz1, bz2)
    for l in range(3):
        aFA0, aFB0 = _sc_edge_pass(gFA, gFB, srcI[0], dstI[0], z128)
        aFA1, aFB1 = _sc_edge_pass(gFA, gFB, srcI[1], dstI[1], z128)
        aCA0, aCB0 = _sc_edge_pass(gCA, gCB, srcI[0], dstI[0], z128)
        aCA1, aCB1 = _sc_edge_pass(gCA, gCB, srcI[1], dstI[1], z128)
        aggs = tuple(a[:_N] for a in
                     (aFA0, aFB0, aFA1, aFB1, aCA0, aCB0, aCA1, aCB1))
        args = aggs + (hF, ctrl, cond, hsum, degA, degB,
                       Wf[l], r2(bf[l]), Wc[l], r2(bc[l]), Wz[l], r2(bz[l]))
        if l < 2:
            hF, ctrl, hsum, gFA, gFB, gCA, gCB = _layer_mid(*args)
        else:
            (hsum,) = _layer_last(*args)

    wcls_pad = jnp.pad(Wcls, ((0, 0), (0, 128 - _C)))
    bcls_pad = jnp.pad(bcls, ((0, 128 - _C),)).reshape(1, 128)
    batch3 = batch.reshape(_GRID, 1, _NBROW)
    (outp,) = _readout(hsum, batch3, wcls_pad, bcls_pad)
    return outp[:, :_C]
